# Initial kernel scaffold; baseline (speedup 1.0000x reference)
import jax
import jax.numpy as jnp
from jax.experimental import pallas as pl


def kernel(images, gt_boxes, gt_classes, anchor_boxes):
    raise NotImplementedError("write your pallas kernel here")


# speedup vs baseline: 18.8660x; 18.8660x over previous
import functools

import jax
import jax.numpy as jnp
from jax.experimental import pallas as pl
from jax.experimental.pallas import tpu as pltpu

POS_THRESH = 0.5
NEG_THRESH = 0.4
BACKGROUND_CLASS = -1.0
IGNORE_CLASS = -2.0

_LANES = 128
_BLK_ROWS = 16


def _encode_body(n_gt, anc_ref, gt_ref, cls_ref, out_ref):
    ax = anc_ref[0]
    ay = anc_ref[1]
    aw = anc_ref[2]
    ah = anc_ref[3]
    ax2 = ax + aw
    ay2 = ay + ah
    area_a = aw * ah

    init = (
        jnp.full_like(ax, -1.0),
        jnp.zeros_like(ax),
        jnp.zeros_like(ax),
        jnp.zeros_like(ax),
        jnp.zeros_like(ax),
        jnp.zeros_like(ax),
    )

    def step(n, carry):
        best, mx, my, mw, mh, mc = carry
        gx = gt_ref[0, 0, 4 * n]
        gy = gt_ref[0, 0, 4 * n + 1]
        gw = gt_ref[0, 0, 4 * n + 2]
        gh = gt_ref[0, 0, 4 * n + 3]
        gc = cls_ref[0, 0, n]
        gx2 = gx + gw
        gy2 = gy + gh
        area_g = gw * gh
        iw = jnp.maximum(jnp.minimum(ax2, gx2) - jnp.maximum(ax, gx), 0.0)
        ih = jnp.maximum(jnp.minimum(ay2, gy2) - jnp.maximum(ay, gy), 0.0)
        inter = iw * ih
        union = area_a + area_g - inter
        iou = inter / union
        p = iou > best
        return (
            jnp.where(p, iou, best),
            jnp.where(p, gx, mx),
            jnp.where(p, gy, my),
            jnp.where(p, gw, mw),
            jnp.where(p, gh, mh),
            jnp.where(p, gc, mc),
        )

    best, mx, my, mw, mh, mc = jax.lax.fori_loop(0, n_gt, step, init, unroll=4)

    acx = ax + aw * 0.5
    acy = ay + ah * 0.5
    bcx = mx + mw * 0.5
    bcy = my + mh * 0.5
    d0 = ((bcx - acx) / aw) / 0.1
    d1 = ((bcy - acy) / ah) / 0.1
    d2 = jnp.log(mw / aw) / 0.2
    d3 = jnp.log(mh / ah) / 0.2
    pos = best >= POS_THRESH
    ign = jnp.logical_and(best >= NEG_THRESH, jnp.logical_not(pos))
    cls_t = jnp.where(pos, mc, BACKGROUND_CLASS)
    cls_t = jnp.where(ign, IGNORE_CLASS, cls_t)
    nan = (d0 != d0) | (d1 != d1) | (d2 != d2) | (d3 != d3)
    out_ref[0, 0] = jnp.where(nan, IGNORE_CLASS, d0)
    out_ref[0, 1] = jnp.where(nan, IGNORE_CLASS, d1)
    out_ref[0, 2] = jnp.where(nan, IGNORE_CLASS, d2)
    out_ref[0, 3] = jnp.where(nan, IGNORE_CLASS, d3)
    out_ref[0, 4] = jnp.where(nan, IGNORE_CLASS, cls_t)


def kernel(images, gt_boxes, gt_classes, anchor_boxes):
    del images
    B, N = gt_classes.shape
    A = anchor_boxes.shape[0]
    chunk = _BLK_ROWS * _LANES
    a_pad = -(-A // chunk) * chunk
    rows = a_pad // _LANES
    n_blocks = rows // _BLK_ROWS

    anc = anchor_boxes.astype(jnp.float32)
    if a_pad != A:
        pad_box = jnp.array([1e6, 1e6, 16.0, 16.0], dtype=jnp.float32)
        anc = jnp.concatenate(
            [anc, jnp.broadcast_to(pad_box, (a_pad - A, 4))], axis=0
        )
    planes = anc.T.reshape(4, rows, _LANES)
    gt_flat = gt_boxes.astype(jnp.float32).reshape(B, 1, N * 4)
    cls_in = gt_classes.astype(jnp.float32).reshape(B, 1, N)

    out = pl.pallas_call(
        functools.partial(_encode_body, N),
        grid=(B, n_blocks),
        in_specs=[
            pl.BlockSpec((4, _BLK_ROWS, _LANES), lambda b, k: (0, k, 0)),
            pl.BlockSpec(
                (1, 1, N * 4), lambda b, k: (b, 0, 0), memory_space=pltpu.SMEM
            ),
            pl.BlockSpec(
                (1, 1, N), lambda b, k: (b, 0, 0), memory_space=pltpu.SMEM
            ),
        ],
        out_specs=pl.BlockSpec(
            (1, 5, _BLK_ROWS, _LANES), lambda b, k: (b, 0, k, 0)
        ),
        out_shape=jax.ShapeDtypeStruct((B, 5, rows, _LANES), jnp.float32),
    )(planes, gt_flat, cls_in)

    flat = out.reshape(B, 5, a_pad)[:, :, :A]
    box_targets = flat[:, :4].transpose(0, 2, 1)
    class_targets = flat[:, 4]
    return box_targets, class_targets
